# baseline (speedup 1.0000x reference)
import jax
import jax.numpy as jnp
from jax import lax
from jax.experimental import pallas as pl
from jax.experimental.pallas import tpu as pltpu
from jax.experimental.pallas import tpu_sc as plsc

N = 1985
N_MLLM = 1000
N_BYT5 = 256
NCH1 = 80
PAD = 2048
NCH = PAD // 16


def _partition_body(mllm_hbm, byt5_hbm, idx_hbm, zmask_hbm,
                    comb_v, idx_v, zmask_v, sem1, sem2, sem3, sem4):
    @pl.when((lax.axis_index("c") == 0) & (lax.axis_index("s") == 0))
    def _():
        zeros16 = jnp.zeros((16,), jnp.float32)
        comb_v[pl.ds(1248, 16)] = zeros16
        comb_v[pl.ds(1264, 16)] = zeros16
        cp1 = pltpu.make_async_copy(mllm_hbm, comb_v.at[pl.ds(0, N_MLLM)], sem1)
        cp2 = pltpu.make_async_copy(byt5_hbm, comb_v.at[pl.ds(N_MLLM, N_BYT5)], sem2)
        cp1.start()
        cp2.start()

        @plsc.parallel_loop(NCH1, NCH, unroll=4)
        def _(j):
            pos = lax.iota(jnp.int32, 16) + j * 16
            idx_v[pl.ds(j * 16, 16)] = pos
            zmask_v[pl.ds(j * 16, 16)] = zeros16

        cp3 = pltpu.make_async_copy(
            idx_v.at[pl.ds(NCH1 * 16, N - NCH1 * 16)],
            idx_hbm.at[pl.ds(NCH1 * 16, N - NCH1 * 16)], sem3)
        cp4 = pltpu.make_async_copy(
            zmask_v.at[pl.ds(NCH1 * 16, N - NCH1 * 16)],
            zmask_hbm.at[pl.ds(NCH1 * 16, N - NCH1 * 16)], sem4)
        cp3.start()
        cp4.start()

        cp1.wait()
        cp2.wait()

        ones16 = jnp.full((16,), 1.0, jnp.float32)
        iota16 = lax.iota(jnp.int32, 16)

        @plsc.parallel_loop(
            0, NCH1, unroll=4,
            carry=(jnp.zeros((16,), jnp.int32), jnp.zeros((16,), jnp.int32)))
        def _cc(j, carry):
            c1, c0 = carry
            jb = NCH1 - 1 - j
            va = comb_v[pl.ds(j * 16, 16)]
            vb = comb_v[pl.ds(jb * 16, 16)]
            m1 = va == 1.0
            m0 = vb == 0.0
            i1 = m1.astype(jnp.int32)
            i0 = m0.astype(jnp.int32)
            pop1 = plsc.all_reduce_population_count(m1)
            pop0 = plsc.all_reduce_population_count(m0)
            dest1 = c1 + plsc.cumsum(i1) - i1
            dest0 = (NCH1 * 16) - (c0 + pop0 - plsc.cumsum(i0) + i0)
            plsc.store_scatter(idx_v, [dest1], iota16 + j * 16, mask=m1)
            plsc.store_scatter(idx_v, [dest0], iota16 + jb * 16, mask=m0)
            plsc.store_scatter(zmask_v, [dest1], ones16, mask=m1)
            plsc.store_scatter(zmask_v, [dest0], zeros16, mask=m0)
            return (c1 + pop1, c0 + pop0)

        cp5 = pltpu.make_async_copy(
            idx_v.at[pl.ds(0, NCH1 * 16)], idx_hbm.at[pl.ds(0, NCH1 * 16)], sem1)
        cp6 = pltpu.make_async_copy(
            zmask_v.at[pl.ds(0, NCH1 * 16)], zmask_hbm.at[pl.ds(0, NCH1 * 16)],
            sem2)
        cp5.start()
        cp6.start()
        cp5.wait()
        cp6.wait()
        cp3.wait()
        cp4.wait()


_partition = pl.kernel(
    _partition_body,
    out_type=(
        jax.ShapeDtypeStruct((N,), jnp.int32),
        jax.ShapeDtypeStruct((N,), jnp.float32),
    ),
    mesh=plsc.VectorSubcoreMesh(
        core_axis_name="c", subcore_axis_name="s", num_cores=1),
    compiler_params=pltpu.CompilerParams(needs_layout_passes=False),
    scratch_types=[
        pltpu.VMEM((PAD,), jnp.float32),
        pltpu.VMEM((PAD,), jnp.int32),
        pltpu.VMEM((PAD,), jnp.float32),
        pltpu.SemaphoreType.DMA,
        pltpu.SemaphoreType.DMA,
        pltpu.SemaphoreType.DMA,
        pltpu.SemaphoreType.DMA,
    ],
)


@jax.jit
def kernel(mllm_mask, byt5_mask):
    return _partition(mllm_mask, byt5_mask)


# speedup vs baseline: 1.0081x; 1.0081x over previous
import jax
import jax.numpy as jnp
from jax import lax
from jax.experimental import pallas as pl
from jax.experimental.pallas import tpu as pltpu
from jax.experimental.pallas import tpu_sc as plsc

N = 1985
N_MLLM = 1000
N_BYT5 = 256
NCH1 = 80
PAD = 2048
NCH = PAD // 16


def _partition_body(mllm_hbm, byt5_hbm, idx_hbm, zmask_hbm,
                    comb_v, idx_v, zmask_v, sem1, sem2, sem3, sem4):
    @pl.when(lax.axis_index("s") == 0)
    def _():
        zeros16 = jnp.zeros((16,), jnp.float32)
        cp1 = pltpu.make_async_copy(mllm_hbm, comb_v.at[pl.ds(0, N_MLLM)], sem1)
        cp1.start()
        comb_v[pl.ds(1248, 16)] = zeros16
        comb_v[pl.ds(1264, 16)] = zeros16
        cp2 = pltpu.make_async_copy(byt5_hbm, comb_v.at[pl.ds(N_MLLM, N_BYT5)], sem2)
        cp2.start()

        @plsc.parallel_loop(NCH1, NCH, unroll=4)
        def _(j):
            pos = lax.iota(jnp.int32, 16) + j * 16
            idx_v[pl.ds(j * 16, 16)] = pos
            zmask_v[pl.ds(j * 16, 16)] = zeros16

        cp3 = pltpu.make_async_copy(
            idx_v.at[pl.ds(NCH1 * 16, N - NCH1 * 16)],
            idx_hbm.at[pl.ds(NCH1 * 16, N - NCH1 * 16)], sem3)
        cp4 = pltpu.make_async_copy(
            zmask_v.at[pl.ds(NCH1 * 16, N - NCH1 * 16)],
            zmask_hbm.at[pl.ds(NCH1 * 16, N - NCH1 * 16)], sem4)
        cp3.start()
        cp4.start()

        cp1.wait()
        cp2.wait()

        @plsc.parallel_loop(0, NCH1, unroll=4, carry=zeros16)
        def acc(j, a):
            return a + comb_v[pl.ds(j * 16, 16)]

        nv = jnp.sum(acc, axis=0).astype(jnp.int32)

        @plsc.parallel_loop(0, NCH1, unroll=4, carry=jnp.zeros((16,), jnp.int32))
        def _c1(j, c1_vec):
            v = comb_v[pl.ds(j * 16, 16)]
            pos = lax.iota(jnp.int32, 16) + j * 16
            m1 = v == 1.0
            m1_i = m1.astype(jnp.int32)
            pop = plsc.all_reduce_population_count(m1)
            ones_before = c1_vec + plsc.cumsum(m1_i) - m1_i
            dest = jnp.where(m1, ones_before, nv + pos - ones_before)
            plsc.store_scatter(idx_v, [dest], pos)
            zmask_v[pl.ds(j * 16, 16)] = jnp.where(
                pos < nv, jnp.float32(1.0), jnp.float32(0.0)
            )
            return c1_vec + pop

        cp5 = pltpu.make_async_copy(
            idx_v.at[pl.ds(0, NCH1 * 16)], idx_hbm.at[pl.ds(0, NCH1 * 16)], sem1)
        cp6 = pltpu.make_async_copy(
            zmask_v.at[pl.ds(0, NCH1 * 16)], zmask_hbm.at[pl.ds(0, NCH1 * 16)],
            sem2)
        cp5.start()
        cp6.start()
        cp5.wait()
        cp6.wait()
        cp3.wait()
        cp4.wait()


_partition = pl.kernel(
    _partition_body,
    out_type=(
        jax.ShapeDtypeStruct((N,), jnp.int32),
        jax.ShapeDtypeStruct((N,), jnp.float32),
    ),
    mesh=plsc.VectorSubcoreMesh(
        core_axis_name="c", subcore_axis_name="s", num_cores=1),
    compiler_params=pltpu.CompilerParams(needs_layout_passes=False),
    scratch_types=[
        pltpu.VMEM((PAD,), jnp.float32),
        pltpu.VMEM((PAD,), jnp.int32),
        pltpu.VMEM((PAD,), jnp.float32),
        pltpu.SemaphoreType.DMA,
        pltpu.SemaphoreType.DMA,
        pltpu.SemaphoreType.DMA,
        pltpu.SemaphoreType.DMA,
    ],
)


@jax.jit
def kernel(mllm_mask, byt5_mask):
    return _partition(mllm_mask, byt5_mask)
